# baseline (speedup 1.0000x reference)
import functools

import jax
import jax.numpy as jnp
from jax import lax
from jax.experimental import pallas as pl
from jax.experimental.pallas import tpu as pltpu
from jax.experimental.pallas import tpu_sc as plsc

RC = 5.2

NC, NS, L = 2, 16, 16
NW = NC * NS

CHUNK = 64

MROWS = 10240
RPT = MROWS // NS


def _celu(x):
    return jnp.where(x > 0, x, jnp.exp(x) - 1.0)


def _expert_linear(x, oh, w_ref, b_ref):
    n_sp = w_ref.shape[0]
    acc = None
    for s in range(n_sp):
        y = jnp.dot(x, w_ref[s], preferred_element_type=jnp.float32)
        y = oh[:, s:s + 1] * (y + b_ref[s][None, :])
        acc = y if acc is None else acc + y
    return acc


def _k1_body(x_ref, oh_ref, w0_ref, b0_ref, wn0_ref, bn0_ref,
             int_ref, ngh_ref):
    x = x_ref[...]
    oh = oh_ref[...]
    internal = _celu(_expert_linear(x, oh, w0_ref, b0_ref))
    int_ref[...] = internal
    ngh_ref[...] = _celu(_expert_linear(internal, oh, wn0_ref, bn0_ref))


def _k2_body(x_ref, ma_ref, mb_ref, oh_ref, w1a_ref, w1b_ref, b1_ref,
             wn1_ref, bn1_ref, int_ref, ngh_ref):
    x = x_ref[...]
    m = ma_ref[...] + mb_ref[...]
    oh = oh_ref[...]
    n_sp = w1a_ref.shape[0]
    acc = None
    for s in range(n_sp):
        y = jnp.dot(x, w1a_ref[s], preferred_element_type=jnp.float32)
        y = y + jnp.dot(m, w1b_ref[s], preferred_element_type=jnp.float32)
        y = oh[:, s:s + 1] * (y + b1_ref[s][None, :])
        acc = y if acc is None else acc + y
    internal = _celu(acc)
    int_ref[...] = internal
    ngh_ref[...] = _celu(_expert_linear(internal, oh, wn1_ref, bn1_ref))


def _k3_body(x_ref, ma_ref, mb_ref, oh_ref, sp_ref, wfa_ref, wfb_ref,
             bf_ref, tq_ref, ch_ref, pc_ref):
    x = x_ref[...]
    m = ma_ref[...] + mb_ref[...]
    p = jnp.dot(x, wfa_ref[...], preferred_element_type=jnp.float32)
    p = p + jnp.dot(m, wfb_ref[...], preferred_element_type=jnp.float32)
    p = p + bf_ref[...]
    prech = jnp.sum(oh_ref[...] * p, axis=1)
    b, a = sp_ref.shape
    prech = prech.reshape(b, a)
    sp = sp_ref[...]
    dummy = sp != -1
    cnt = jnp.sum(dummy.astype(jnp.float32), axis=1, keepdims=True)
    tp = jnp.sum(prech, axis=1, keepdims=True)
    ch = prech + (tq_ref[...] - tp) / cnt
    ch_ref[...] = jnp.where(dummy, ch, 0.0)
    pc_ref[...] = prech


def _sc_scatter_build(cpt):
    mesh = plsc.VectorSubcoreMesh(core_axis_name="c", subcore_axis_name="s",
                                  num_cores=NC, num_subcores=NS)

    nbuf = 3

    @functools.partial(
        pl.kernel,
        out_type=jax.ShapeDtypeStruct((NC, MROWS, 128), jnp.float32),
        mesh=mesh,
        scratch_types=[
            pltpu.VMEM((cpt, 2 * CHUNK), jnp.int32),
            [pltpu.VMEM((CHUNK, 128), jnp.float32)] * nbuf,
            [pltpu.VMEM((CHUNK,), jnp.int32)] * nbuf,
            [pltpu.VMEM((CHUNK,), jnp.int32)] * nbuf,
            pltpu.VMEM((CHUNK,), jnp.float32),
            pltpu.VMEM((2, L), jnp.float32),
            pltpu.VMEM_SHARED((MROWS, 128), jnp.float32),
            [pltpu.SemaphoreType.DMA] * nbuf,
            [pltpu.SemaphoreType.DMA] * nbuf,
        ],
        compiler_params=pltpu.CompilerParams(needs_layout_passes=False),
    )
    def sc_kernel(neigh, ipack, dcoef, out,
                  ipack_v, rows, isrc_c, idst_c, dec_v, dcoef_v, acc,
                  gsem, ssem):
        c = lax.axis_index("c")
        s = lax.axis_index("s")
        w = c * NS + s

        def _zrow(r, carry):
            for g in range(8):
                rows[0][r, pl.ds(g * L, L)] = jnp.zeros((L,), jnp.float32)
            return carry
        lax.fori_loop(0, CHUNK, _zrow, 0)
        for k in range(RPT // CHUNK):
            pltpu.sync_copy(rows[0],
                            acc.at[pl.ds(s * RPT + k * CHUNK, CHUNK)])

        pltpu.sync_copy(ipack.at[pl.ds(w * cpt, cpt)], ipack_v)
        pltpu.sync_copy(dcoef, dcoef_v)
        dp2 = dcoef_v[0, :]
        df2 = dcoef_v[1, :]

        plsc.subcore_barrier()

        def _prep_and_fire(j, b):
            for g in range(CHUNK // 32):
                word = ipack_v[j, pl.ds(g * L, L)]
                isrc_c[b][pl.ds(g * 32, L)] = word & 0xFFFF
                isrc_c[b][pl.ds(g * 32 + L, L)] = (
                    lax.shift_right_logical(word, 16))
            for g in range(CHUNK // 32):
                word = ipack_v[j, pl.ds(CHUNK // 2 + g * L, L)]
                idst_c[b][pl.ds(g * 32, L)] = word & 0xFFFF
                idst_c[b][pl.ds(g * 32 + L, L)] = (
                    lax.shift_right_logical(word, 16))
            pltpu.async_copy(neigh.at[isrc_c[b]], rows[b], gsem[b])

        def _do_chunk(j, b, drain_scatter):
            bn = (b + 1) % nbuf

            @pl.when(j + 1 < cpt)
            def _():
                if drain_scatter:
                    pltpu.make_async_copy(
                        rows[bn], acc.at[idst_c[bn]], ssem[bn]).wait()
                _prep_and_fire(j + 1, bn)

            pltpu.make_async_copy(
                neigh.at[isrc_c[b]], rows[b], gsem[b]).wait()
            for g in range(CHUNK // L):
                d = plsc.bitcast(ipack_v[j, pl.ds(CHUNK + g * L, L)],
                                 jnp.float32)
                x = d * (1.0 / RC)
                x2 = jnp.clip(x * x, 0.0, 1.0 - 1e-6)
                f = jnp.exp(1.0 - 1.0 / (1.0 - x2))
                dec = jnp.where(d < RC, f, 0.0)
                dec = dp2 * jnp.exp(-df2 * d) * dec
                dec_v[pl.ds(g * L, L)] = dec
            def _scale8(t, carry2):
                for u in range(8):
                    e = t * 8 + u
                    bc = plsc.load_gather(
                        dec_v, [jnp.full((L,), e, jnp.int32)])
                    for q in range(8):
                        rows[b][e, pl.ds(q * L, L)] = (
                            rows[b][e, pl.ds(q * L, L)] * bc)
                return carry2
            lax.fori_loop(0, CHUNK // 8, _scale8, 0)
            pltpu.async_copy(rows[b], acc.at[idst_c[b]], ssem[b], add=True)

        _prep_and_fire(0, 0)
        for b in range(nbuf):
            _do_chunk(b, b, drain_scatter=(b + 1 >= nbuf))

        def _round(r, carry):
            for b in range(nbuf):
                _do_chunk(r * nbuf + b, b, drain_scatter=True)
            return carry
        lax.fori_loop(1, cpt // nbuf, _round, 0)

        for b in range(nbuf):
            pltpu.make_async_copy(
                rows[b], acc.at[idst_c[b]], ssem[b]).wait()

        plsc.subcore_barrier()
        pltpu.sync_copy(acc.at[pl.ds(s * RPT, RPT)],
                        out.at[c, pl.ds(s * RPT, RPT)])

    return sc_kernel


def kernel(species, aev, atom_index12, distances, total_charges,
           W0, b0, Wn0, bn0, W1, b1, Wn1, bn1, Wf, bf,
           decay_prefactor, decay_factor):
    bsz, na = species.shape
    n = bsz * na
    d_aev = aev.shape[-1]
    n_sp = W0.shape[0]
    m0 = W0.shape[-1]
    nb0 = Wn0.shape[-1]
    m1 = W1.shape[-1]
    nb1 = Wn1.shape[-1]
    p = atom_index12.shape[1]

    species_ = species.reshape(-1)
    feats = aev.reshape(n, d_aev)
    oh = (species_[:, None] == jnp.arange(n_sp, dtype=species_.dtype)[None, :]
          ).astype(jnp.float32)

    idx_dst = atom_index12.reshape(-1).astype(jnp.int32)
    idx_src = atom_index12[::-1].reshape(-1).astype(jnp.int32)
    dist2 = jnp.concatenate([distances, distances])
    p2 = 2 * p
    ept = 24 * CHUNK * -(-p2 // (24 * CHUNK * NW))
    cpt = ept // CHUNK
    pad = NW * ept - p2
    t_chunks = NW * cpt
    src_rows = jnp.concatenate(
        [idx_src, jnp.zeros((pad,), jnp.int32)]
    ).reshape(t_chunks, CHUNK // 32, 2, L)
    dst_rows = jnp.concatenate(
        [idx_dst, jnp.zeros((pad,), jnp.int32)]
    ).reshape(t_chunks, CHUNK // 32, 2, L)
    word_src = (src_rows[:, :, 0, :] | (src_rows[:, :, 1, :] << 16))
    word_dst = (dst_rows[:, :, 0, :] | (dst_rows[:, :, 1, :] << 16))
    dist_bits = jax.lax.bitcast_convert_type(
        jnp.concatenate([dist2, jnp.full((pad,), 1e9, jnp.float32)]
                        ).reshape(t_chunks, CHUNK), jnp.int32)
    ipack = jnp.concatenate([word_src.reshape(t_chunks, CHUNK // 2),
                             word_dst.reshape(t_chunks, CHUNK // 2),
                             dist_bits], axis=1)
    dcoef = jnp.stack([
        jnp.full((L,), decay_prefactor.astype(jnp.float32) ** 2),
        jnp.full((L,), decay_factor.astype(jnp.float32) ** 2)])

    sc_scatter = _sc_scatter_build(cpt)

    rows = 400
    grid = (n // rows,)
    wspec3 = lambda shp: pl.BlockSpec(shp, lambda i: (0, 0, 0))
    wspec2 = lambda shp: pl.BlockSpec(shp, lambda i: (0, 0))
    rspec = lambda width: pl.BlockSpec((rows, width), lambda i: (i, 0))

    k1 = pl.pallas_call(
        _k1_body,
        grid=grid,
        in_specs=[rspec(d_aev), rspec(n_sp),
                  wspec3((n_sp, d_aev, m0)), wspec2((n_sp, m0)),
                  wspec3((n_sp, m0, nb0)), wspec2((n_sp, nb0))],
        out_specs=[rspec(m0), rspec(nb0)],
        out_shape=[jax.ShapeDtypeStruct((n, m0), jnp.float32),
                   jax.ShapeDtypeStruct((n, nb0), jnp.float32)],
    )
    internal0, neigh0 = k1(feats, oh, W0, b0, Wn0, bn0)

    merged0 = sc_scatter(neigh0, ipack, dcoef)

    W1a = W1[:, :m0, :]
    W1b = W1[:, m0:, :]
    mspec = pl.BlockSpec((rows, nb0), lambda i: (i, 0))
    k2 = pl.pallas_call(
        _k2_body,
        grid=grid,
        in_specs=[rspec(m0), mspec, mspec, rspec(n_sp),
                  wspec3((n_sp, m0, m1)), wspec3((n_sp, nb0, m1)),
                  wspec2((n_sp, m1)),
                  wspec3((n_sp, m1, nb1)), wspec2((n_sp, nb1))],
        out_specs=[rspec(m1), rspec(nb1)],
        out_shape=[jax.ShapeDtypeStruct((n, m1), jnp.float32),
                   jax.ShapeDtypeStruct((n, nb1), jnp.float32)],
    )
    internal1, neigh1 = k2(internal0, merged0[0], merged0[1], oh,
                           W1a, W1b, b1, Wn1, bn1)

    merged1 = sc_scatter(neigh1, ipack, dcoef)

    wfa = Wf[:, :m1, 0].T
    wfb = Wf[:, m1:, 0].T
    bfv = bf.reshape(1, n_sp)
    z2 = lambda i: (0, 0)
    k3 = pl.pallas_call(
        _k3_body,
        grid=(1,),
        in_specs=[pl.BlockSpec((n, m1), z2),
                  pl.BlockSpec((n, nb1), z2),
                  pl.BlockSpec((n, nb1), z2),
                  pl.BlockSpec((n, n_sp), z2),
                  pl.BlockSpec((bsz, na), z2),
                  pl.BlockSpec((m1, n_sp), z2),
                  pl.BlockSpec((nb1, n_sp), z2),
                  pl.BlockSpec((1, n_sp), z2),
                  pl.BlockSpec((bsz, 1), z2)],
        out_specs=[pl.BlockSpec((bsz, na), z2),
                   pl.BlockSpec((bsz, na), z2)],
        out_shape=[jax.ShapeDtypeStruct((bsz, na), jnp.float32),
                   jax.ShapeDtypeStruct((bsz, na), jnp.float32)],
    )
    charges, precharges = k3(internal1, merged1[0], merged1[1], oh, species,
                             wfa, wfb, bfv, total_charges.reshape(bsz, 1))
    return species, charges, precharges


# speedup vs baseline: 3.0375x; 3.0375x over previous
import functools

import jax
import jax.numpy as jnp
from jax import lax
from jax.experimental import pallas as pl
from jax.experimental.pallas import tpu as pltpu
from jax.experimental.pallas import tpu_sc as plsc

RC = 5.2

NC, NS, L = 2, 16, 16
NW = NC * NS

CHUNK = 64

MROWS = 10240
RPT = MROWS // NS


def _celu(x):
    return jnp.where(x > 0, x, jnp.exp(x) - 1.0)


def _expert_linear(x, oh, w_ref, b_ref):
    n_sp = w_ref.shape[0]
    acc = None
    for s in range(n_sp):
        y = jnp.dot(x, w_ref[s], preferred_element_type=jnp.float32)
        y = oh[:, s:s + 1] * (y + b_ref[s][None, :])
        acc = y if acc is None else acc + y
    return acc


def _k1_body(x_ref, oh_ref, w0_ref, b0_ref, wn0_ref, bn0_ref,
             int_ref, ngh_ref):
    x = x_ref[...]
    oh = oh_ref[...]
    internal = _celu(_expert_linear(x, oh, w0_ref, b0_ref))
    int_ref[...] = internal
    ngh_ref[...] = _celu(_expert_linear(internal, oh, wn0_ref, bn0_ref))


def _k2_body(x_ref, ma_ref, mb_ref, oh_ref, w1a_ref, w1b_ref, b1_ref,
             wn1_ref, bn1_ref, int_ref, ngh_ref):
    x = x_ref[...]
    m = ma_ref[...] + mb_ref[...]
    oh = oh_ref[...]
    n_sp = w1a_ref.shape[0]
    acc = None
    for s in range(n_sp):
        y = jnp.dot(x, w1a_ref[s], preferred_element_type=jnp.float32)
        y = y + jnp.dot(m, w1b_ref[s], preferred_element_type=jnp.float32)
        y = oh[:, s:s + 1] * (y + b1_ref[s][None, :])
        acc = y if acc is None else acc + y
    internal = _celu(acc)
    int_ref[...] = internal
    ngh_ref[...] = _celu(_expert_linear(internal, oh, wn1_ref, bn1_ref))


def _k3_body(x_ref, ma_ref, mb_ref, oh_ref, sp_ref, wfa_ref, wfb_ref,
             bf_ref, tq_ref, ch_ref, pc_ref):
    x = x_ref[...]
    m = ma_ref[...] + mb_ref[...]
    p = jnp.dot(x, wfa_ref[...], preferred_element_type=jnp.float32)
    p = p + jnp.dot(m, wfb_ref[...], preferred_element_type=jnp.float32)
    p = p + bf_ref[...]
    prech = jnp.sum(oh_ref[...] * p, axis=1)
    b, a = sp_ref.shape
    prech = prech.reshape(b, a)
    sp = sp_ref[...]
    dummy = sp != -1
    cnt = jnp.sum(dummy.astype(jnp.float32), axis=1, keepdims=True)
    tp = jnp.sum(prech, axis=1, keepdims=True)
    ch = prech + (tq_ref[...] - tp) / cnt
    ch_ref[...] = jnp.where(dummy, ch, 0.0)
    pc_ref[...] = prech


def _sc_scatter_build(cpt):
    mesh = plsc.VectorSubcoreMesh(core_axis_name="c", subcore_axis_name="s",
                                  num_cores=NC, num_subcores=NS)

    nbuf = 3

    @functools.partial(
        pl.kernel,
        out_type=jax.ShapeDtypeStruct((NC, MROWS, 128), jnp.float32),
        mesh=mesh,
        scratch_types=[
            pltpu.VMEM((cpt, 2 * CHUNK), jnp.int32),
            [pltpu.VMEM((CHUNK, 128), jnp.float32)] * nbuf,
            [pltpu.VMEM((CHUNK,), jnp.int32)] * nbuf,
            [pltpu.VMEM((CHUNK,), jnp.int32)] * nbuf,
            pltpu.VMEM((CHUNK,), jnp.float32),
            pltpu.VMEM((2, L), jnp.float32),
            pltpu.VMEM_SHARED((MROWS, 128), jnp.float32),
            [pltpu.SemaphoreType.DMA] * nbuf,
            [pltpu.SemaphoreType.DMA] * nbuf,
        ],
        compiler_params=pltpu.CompilerParams(needs_layout_passes=False),
    )
    def sc_kernel(neigh, ipack, dcoef, out,
                  ipack_v, rows, isrc_c, idst_c, dec_v, dcoef_v, acc,
                  gsem, ssem):
        c = lax.axis_index("c")
        s = lax.axis_index("s")
        w = c * NS + s

        def _zrow(r, carry):
            for g in range(8):
                rows[0][r, pl.ds(g * L, L)] = jnp.zeros((L,), jnp.float32)
            return carry
        lax.fori_loop(0, CHUNK, _zrow, 0)
        for k in range(RPT // CHUNK):
            pltpu.sync_copy(rows[0],
                            acc.at[pl.ds(s * RPT + k * CHUNK, CHUNK)])

        pltpu.sync_copy(ipack.at[pl.ds(w * cpt, cpt)], ipack_v)
        pltpu.sync_copy(dcoef, dcoef_v)
        dp2 = dcoef_v[0, :]
        df2 = dcoef_v[1, :]

        plsc.subcore_barrier()

        def _prep_and_fire(j, b):
            for g in range(CHUNK // 32):
                word = ipack_v[j, pl.ds(g * L, L)]
                isrc_c[b][pl.ds(g * 32, L)] = word & 0xFFFF
                isrc_c[b][pl.ds(g * 32 + L, L)] = (
                    lax.shift_right_logical(word, 16))
            for g in range(CHUNK // 32):
                word = ipack_v[j, pl.ds(CHUNK // 2 + g * L, L)]
                idst_c[b][pl.ds(g * 32, L)] = word & 0xFFFF
                idst_c[b][pl.ds(g * 32 + L, L)] = (
                    lax.shift_right_logical(word, 16))
            pltpu.async_copy(neigh.at[isrc_c[b]], rows[b], gsem[b])

        def _do_chunk(j, b, drain_scatter):
            bn = (b + 1) % nbuf

            @pl.when(j + 1 < cpt)
            def _():
                if drain_scatter:
                    pltpu.make_async_copy(
                        rows[bn], acc.at[idst_c[bn]], ssem[bn]).wait()
                _prep_and_fire(j + 1, bn)

            pltpu.make_async_copy(
                neigh.at[isrc_c[b]], rows[b], gsem[b]).wait()
            for g in range(CHUNK // L):
                d = plsc.bitcast(ipack_v[j, pl.ds(CHUNK + g * L, L)],
                                 jnp.float32)
                x = d * (1.0 / RC)
                x2 = jnp.clip(x * x, 0.0, 1.0 - 1e-6)
                f = jnp.exp(1.0 - 1.0 / (1.0 - x2))
                dec = jnp.where(d < RC, f, 0.0)
                dec = dp2 * jnp.exp(-df2 * d) * dec
                dec_v[pl.ds(g * L, L)] = dec
            def _scale8(t, carry2):
                for u in range(8):
                    e = t * 8 + u
                    bc = plsc.load_gather(
                        dec_v, [jnp.full((L,), e, jnp.int32)])
                    for q in range(8):
                        rows[b][e, pl.ds(q * L, L)] = (
                            rows[b][e, pl.ds(q * L, L)] * bc)
                return carry2
            lax.fori_loop(0, CHUNK // 8, _scale8, 0)
            pltpu.async_copy(rows[b], acc.at[idst_c[b]], ssem[b], add=True)

        _prep_and_fire(0, 0)
        for b in range(nbuf):
            _do_chunk(b, b, drain_scatter=(b + 1 >= nbuf))

        def _round(r, carry):
            for b in range(nbuf):
                _do_chunk(r * nbuf + b, b, drain_scatter=True)
            return carry
        lax.fori_loop(1, cpt // nbuf, _round, 0)

        for b in range(nbuf):
            pltpu.make_async_copy(
                rows[b], acc.at[idst_c[b]], ssem[b]).wait()

        plsc.subcore_barrier()
        pltpu.sync_copy(acc.at[pl.ds(s * RPT, RPT)],
                        out.at[c, pl.ds(s * RPT, RPT)])

    return sc_kernel


def kernel(species, aev, atom_index12, distances, total_charges,
           W0, b0, Wn0, bn0, W1, b1, Wn1, bn1, Wf, bf,
           decay_prefactor, decay_factor):
    bsz, na = species.shape
    n = bsz * na
    d_aev = aev.shape[-1]
    n_sp = W0.shape[0]
    m0 = W0.shape[-1]
    nb0 = Wn0.shape[-1]
    m1 = W1.shape[-1]
    nb1 = Wn1.shape[-1]
    p = atom_index12.shape[1]

    species_ = species.reshape(-1)
    feats = aev.reshape(n, d_aev)
    oh = (species_[:, None] == jnp.arange(n_sp, dtype=species_.dtype)[None, :]
          ).astype(jnp.float32)

    idx_dst = atom_index12.reshape(-1).astype(jnp.int32)
    idx_src = atom_index12[::-1].reshape(-1).astype(jnp.int32)
    dist2 = jnp.concatenate([distances, distances])
    p2 = 2 * p
    ept = 24 * CHUNK * -(-p2 // (24 * CHUNK * NW))
    cpt = ept // CHUNK
    pad = NW * ept - p2
    t_chunks = NW * cpt
    ar = jnp.arange(pad, dtype=jnp.int32)
    pad_src = (ar * 97) % n
    pad_dst = n + (ar % (MROWS - n))
    src_rows = jnp.concatenate(
        [idx_src, pad_src]).reshape(t_chunks, CHUNK // 32, 2, L)
    dst_rows = jnp.concatenate(
        [idx_dst, pad_dst]).reshape(t_chunks, CHUNK // 32, 2, L)
    word_src = (src_rows[:, :, 0, :] | (src_rows[:, :, 1, :] << 16))
    word_dst = (dst_rows[:, :, 0, :] | (dst_rows[:, :, 1, :] << 16))
    dist_bits = jax.lax.bitcast_convert_type(
        jnp.concatenate([dist2, jnp.full((pad,), 1e9, jnp.float32)]
                        ).reshape(t_chunks, CHUNK), jnp.int32)
    ipack = jnp.concatenate([word_src.reshape(t_chunks, CHUNK // 2),
                             word_dst.reshape(t_chunks, CHUNK // 2),
                             dist_bits], axis=1)
    dcoef = jnp.stack([
        jnp.full((L,), decay_prefactor.astype(jnp.float32) ** 2),
        jnp.full((L,), decay_factor.astype(jnp.float32) ** 2)])

    sc_scatter = _sc_scatter_build(cpt)

    rows = 400
    grid = (n // rows,)
    wspec3 = lambda shp: pl.BlockSpec(shp, lambda i: (0, 0, 0))
    wspec2 = lambda shp: pl.BlockSpec(shp, lambda i: (0, 0))
    rspec = lambda width: pl.BlockSpec((rows, width), lambda i: (i, 0))

    k1 = pl.pallas_call(
        _k1_body,
        grid=grid,
        in_specs=[rspec(d_aev), rspec(n_sp),
                  wspec3((n_sp, d_aev, m0)), wspec2((n_sp, m0)),
                  wspec3((n_sp, m0, nb0)), wspec2((n_sp, nb0))],
        out_specs=[rspec(m0), rspec(nb0)],
        out_shape=[jax.ShapeDtypeStruct((n, m0), jnp.float32),
                   jax.ShapeDtypeStruct((n, nb0), jnp.float32)],
    )
    internal0, neigh0 = k1(feats, oh, W0, b0, Wn0, bn0)

    merged0 = sc_scatter(neigh0, ipack, dcoef)

    W1a = W1[:, :m0, :]
    W1b = W1[:, m0:, :]
    mspec = pl.BlockSpec((rows, nb0), lambda i: (i, 0))
    k2 = pl.pallas_call(
        _k2_body,
        grid=grid,
        in_specs=[rspec(m0), mspec, mspec, rspec(n_sp),
                  wspec3((n_sp, m0, m1)), wspec3((n_sp, nb0, m1)),
                  wspec2((n_sp, m1)),
                  wspec3((n_sp, m1, nb1)), wspec2((n_sp, nb1))],
        out_specs=[rspec(m1), rspec(nb1)],
        out_shape=[jax.ShapeDtypeStruct((n, m1), jnp.float32),
                   jax.ShapeDtypeStruct((n, nb1), jnp.float32)],
    )
    internal1, neigh1 = k2(internal0, merged0[0], merged0[1], oh,
                           W1a, W1b, b1, Wn1, bn1)

    merged1 = sc_scatter(neigh1, ipack, dcoef)

    wfa = Wf[:, :m1, 0].T
    wfb = Wf[:, m1:, 0].T
    bfv = bf.reshape(1, n_sp)
    z2 = lambda i: (0, 0)
    k3 = pl.pallas_call(
        _k3_body,
        grid=(1,),
        in_specs=[pl.BlockSpec((n, m1), z2),
                  pl.BlockSpec((n, nb1), z2),
                  pl.BlockSpec((n, nb1), z2),
                  pl.BlockSpec((n, n_sp), z2),
                  pl.BlockSpec((bsz, na), z2),
                  pl.BlockSpec((m1, n_sp), z2),
                  pl.BlockSpec((nb1, n_sp), z2),
                  pl.BlockSpec((1, n_sp), z2),
                  pl.BlockSpec((bsz, 1), z2)],
        out_specs=[pl.BlockSpec((bsz, na), z2),
                   pl.BlockSpec((bsz, na), z2)],
        out_shape=[jax.ShapeDtypeStruct((bsz, na), jnp.float32),
                   jax.ShapeDtypeStruct((bsz, na), jnp.float32)],
    )
    charges, precharges = k3(internal1, merged1[0], merged1[1], oh, species,
                             wfa, wfb, bfv, total_charges.reshape(bsz, 1))
    return species, charges, precharges


# speedup vs baseline: 7.2109x; 2.3739x over previous
import functools

import jax
import jax.numpy as jnp
from jax import lax
from jax.experimental import pallas as pl
from jax.experimental.pallas import tpu as pltpu
from jax.experimental.pallas import tpu_sc as plsc

RC = 5.2

NC, NS, L = 2, 16, 16
NW = NC * NS

CHUNK = 80

MROWS = 10240
RPT = MROWS // NS


def _celu(x):
    return jnp.where(x > 0, x, jnp.exp(x) - 1.0)


def _expert_linear(x, oh, w_ref, b_ref):
    n_sp = w_ref.shape[0]
    acc = None
    for s in range(n_sp):
        y = jnp.dot(x, w_ref[s], preferred_element_type=jnp.float32)
        y = oh[:, s:s + 1] * (y + b_ref[s][None, :])
        acc = y if acc is None else acc + y
    return acc


def _k1_body(x_ref, oh_ref, w0_ref, b0_ref, wn0_ref, bn0_ref,
             int_ref, ngh_ref):
    x = x_ref[...]
    oh = oh_ref[...]
    internal = _celu(_expert_linear(x, oh, w0_ref, b0_ref))
    int_ref[...] = internal
    ngh_ref[...] = _celu(_expert_linear(internal, oh, wn0_ref, bn0_ref))


def _k2_body(x_ref, ma_ref, mb_ref, oh_ref, w1a_ref, w1b_ref, b1_ref,
             wn1_ref, bn1_ref, int_ref, ngh_ref):
    x = x_ref[...]
    m = ma_ref[...] + mb_ref[...]
    oh = oh_ref[...]
    n_sp = w1a_ref.shape[0]
    acc = None
    for s in range(n_sp):
        y = jnp.dot(x, w1a_ref[s], preferred_element_type=jnp.float32)
        y = y + jnp.dot(m, w1b_ref[s], preferred_element_type=jnp.float32)
        y = oh[:, s:s + 1] * (y + b1_ref[s][None, :])
        acc = y if acc is None else acc + y
    internal = _celu(acc)
    int_ref[...] = internal
    ngh_ref[...] = _celu(_expert_linear(internal, oh, wn1_ref, bn1_ref))


def _k3_body(x_ref, ma_ref, mb_ref, oh_ref, sp_ref, wfa_ref, wfb_ref,
             bf_ref, tq_ref, ch_ref, pc_ref):
    x = x_ref[...]
    m = ma_ref[...] + mb_ref[...]
    p = jnp.dot(x, wfa_ref[...], preferred_element_type=jnp.float32)
    p = p + jnp.dot(m, wfb_ref[...], preferred_element_type=jnp.float32)
    p = p + bf_ref[...]
    prech = jnp.sum(oh_ref[...] * p, axis=1)
    b, a = sp_ref.shape
    prech = prech.reshape(b, a)
    sp = sp_ref[...]
    dummy = sp != -1
    cnt = jnp.sum(dummy.astype(jnp.float32), axis=1, keepdims=True)
    tp = jnp.sum(prech, axis=1, keepdims=True)
    ch = prech + (tq_ref[...] - tp) / cnt
    ch_ref[...] = jnp.where(dummy, ch, 0.0)
    pc_ref[...] = prech


def _sc_scatter_build(p):
    mesh = plsc.VectorSubcoreMesh(core_axis_name="c", subcore_axis_name="s",
                                  num_cores=NC, num_subcores=NS)
    ept = p // NS
    cpt = ept // CHUNK
    assert ept % CHUNK == 0 and CHUNK % 8 == 0 and ept % 8 == 0
    nbuf = 4

    @functools.partial(
        pl.kernel,
        out_type=jax.ShapeDtypeStruct((NC, MROWS, 128), jnp.float32),
        mesh=mesh,
        scratch_types=[
            [pltpu.VMEM((CHUNK, 128), jnp.float32)] * nbuf,
            [pltpu.VMEM((CHUNK,), jnp.int32)] * nbuf,
            [pltpu.VMEM((CHUNK,), jnp.int32)] * nbuf,
            [pltpu.VMEM((CHUNK,), jnp.float32)] * nbuf,
            pltpu.VMEM((2, L), jnp.float32),
            pltpu.VMEM_SHARED((MROWS, 128), jnp.float32),
            [pltpu.SemaphoreType.DMA] * nbuf,
            [pltpu.SemaphoreType.DMA] * nbuf,
            [pltpu.SemaphoreType.DMA] * nbuf,
        ],
        compiler_params=pltpu.CompilerParams(needs_layout_passes=False),
    )
    def sc_kernel(neigh, idx, dist, dcoef, out,
                  rows, isrc_c, idst_c, dist_c, dcoef_v, acc,
                  isem, gsem, ssem):
        c = lax.axis_index("c")
        s = lax.axis_index("s")
        dst_base = c * p + s * ept
        src_base = (1 - c) * p + s * ept
        d_base = s * ept

        def _zrow(r, carry):
            for g in range(8):
                rows[0][r, pl.ds(g * L, L)] = jnp.zeros((L,), jnp.float32)
            return carry
        lax.fori_loop(0, CHUNK, _zrow, 0)
        for k in range(RPT // CHUNK):
            pltpu.sync_copy(rows[0],
                            acc.at[pl.ds(s * RPT + k * CHUNK, CHUNK)])

        pltpu.sync_copy(dcoef, dcoef_v)
        dp2 = dcoef_v[0, :]
        df2 = dcoef_v[1, :]

        plsc.subcore_barrier()

        def _fire_idx(j, sl):
            off = j * CHUNK
            pltpu.async_copy(idx.at[pl.ds(src_base + off, CHUNK)],
                             isrc_c[sl], isem[sl])
            pltpu.async_copy(idx.at[pl.ds(dst_base + off, CHUNK)],
                             idst_c[sl], isem[sl])
            pltpu.async_copy(dist.at[pl.ds(d_base + off, CHUNK)],
                             dist_c[sl], isem[sl])

        def _wait_idx(j, sl):
            off = j * CHUNK
            pltpu.make_async_copy(idx.at[pl.ds(src_base + off, CHUNK)],
                                  isrc_c[sl], isem[sl]).wait()
            pltpu.make_async_copy(idx.at[pl.ds(dst_base + off, CHUNK)],
                                  idst_c[sl], isem[sl]).wait()
            pltpu.make_async_copy(dist.at[pl.ds(d_base + off, CHUNK)],
                                  dist_c[sl], isem[sl]).wait()


        def _make_step(sl_proc, sl_g, sl_i):
            def step(j, drain, fetch, fire):
                if fetch:
                    @pl.when(j + 2 < cpt)
                    def _():
                        if drain:
                            pltpu.make_async_copy(
                                rows[sl_i], acc.at[idst_c[sl_i]],
                                ssem[sl_i]).wait()
                        _fire_idx(j + 2, sl_i)
                if fire:
                    @pl.when(j + 1 < cpt)
                    def _():
                        _wait_idx(j + 1, sl_g)
                        pltpu.async_copy(neigh.at[isrc_c[sl_g]],
                                         rows[sl_g], gsem[sl_g])
                pltpu.make_async_copy(neigh.at[isrc_c[sl_proc]],
                                      rows[sl_proc], gsem[sl_proc]).wait()
                for g in range(CHUNK // L):
                    d = dist_c[sl_proc][pl.ds(g * L, L)]
                    x = d * (1.0 / RC)
                    x2 = jnp.clip(x * x, 0.0, 1.0 - 1e-6)
                    f = jnp.exp(1.0 - 1.0 / (1.0 - x2))
                    dec = jnp.where(d < RC, f, 0.0)
                    dist_c[sl_proc][pl.ds(g * L, L)] = (
                        dp2 * jnp.exp(-df2 * d) * dec)
                def _scale8(t, carry2):
                    for u in range(8):
                        e = t * 8 + u
                        bc = plsc.load_gather(
                            dist_c[sl_proc], [jnp.full((L,), e, jnp.int32)])
                        for q in range(8):
                            rows[sl_proc][e, pl.ds(q * L, L)] = (
                                rows[sl_proc][e, pl.ds(q * L, L)] * bc)
                    return carry2
                lax.fori_loop(0, CHUNK // 8, _scale8, 0)
                pltpu.async_copy(rows[sl_proc], acc.at[idst_c[sl_proc]],
                                 ssem[sl_proc], add=True)
            return step

        steps = [_make_step(b, (b + 1) % nbuf, (b + 2) % nbuf)
                 for b in range(nbuf)]

        _fire_idx(0, 0)
        _fire_idx(1, 1)
        _wait_idx(0, 0)
        pltpu.async_copy(neigh.at[isrc_c[0]], rows[0], gsem[0])
        for b in range(nbuf):
            steps[b](b, drain=(b >= 2), fetch=True, fire=True)

        def _round(r, carry):
            for b in range(nbuf):
                steps[b](r * nbuf + b, drain=True, fetch=True, fire=True)
            return carry
        lax.fori_loop(1, cpt // nbuf, _round, 0)

        for t in range(cpt - (cpt // nbuf) * nbuf):
            j = (cpt // nbuf) * nbuf + t
            steps[j % nbuf](j, drain=True, fetch=True, fire=True)
        for b in range(nbuf):
            pltpu.make_async_copy(
                rows[b], acc.at[idst_c[b]], ssem[b]).wait()

        plsc.subcore_barrier()
        pltpu.sync_copy(acc.at[pl.ds(s * RPT, RPT)],
                        out.at[c, pl.ds(s * RPT, RPT)])

    return sc_kernel


def kernel(species, aev, atom_index12, distances, total_charges,
           W0, b0, Wn0, bn0, W1, b1, Wn1, bn1, Wf, bf,
           decay_prefactor, decay_factor):
    bsz, na = species.shape
    n = bsz * na
    d_aev = aev.shape[-1]
    n_sp = W0.shape[0]
    m0 = W0.shape[-1]
    nb0 = Wn0.shape[-1]
    m1 = W1.shape[-1]
    nb1 = Wn1.shape[-1]
    p = atom_index12.shape[1]

    species_ = species.reshape(-1)
    feats = aev.reshape(n, d_aev)
    oh = (species_[:, None] == jnp.arange(n_sp, dtype=species_.dtype)[None, :]
          ).astype(jnp.float32)

    idx_flat = atom_index12.reshape(-1).astype(jnp.int32)
    dcoef = jnp.stack([
        jnp.full((L,), decay_prefactor.astype(jnp.float32) ** 2),
        jnp.full((L,), decay_factor.astype(jnp.float32) ** 2)])

    sc_scatter = _sc_scatter_build(p)

    rows = 400
    grid = (n // rows,)
    wspec3 = lambda shp: pl.BlockSpec(shp, lambda i: (0, 0, 0))
    wspec2 = lambda shp: pl.BlockSpec(shp, lambda i: (0, 0))
    rspec = lambda width: pl.BlockSpec((rows, width), lambda i: (i, 0))

    k1 = pl.pallas_call(
        _k1_body,
        grid=grid,
        in_specs=[rspec(d_aev), rspec(n_sp),
                  wspec3((n_sp, d_aev, m0)), wspec2((n_sp, m0)),
                  wspec3((n_sp, m0, nb0)), wspec2((n_sp, nb0))],
        out_specs=[rspec(m0), rspec(nb0)],
        out_shape=[jax.ShapeDtypeStruct((n, m0), jnp.float32),
                   jax.ShapeDtypeStruct((n, nb0), jnp.float32)],
    )
    internal0, neigh0 = k1(feats, oh, W0, b0, Wn0, bn0)

    merged0 = sc_scatter(neigh0, idx_flat, distances, dcoef)

    W1a = W1[:, :m0, :]
    W1b = W1[:, m0:, :]
    mspec = pl.BlockSpec((rows, nb0), lambda i: (i, 0))
    k2 = pl.pallas_call(
        _k2_body,
        grid=grid,
        in_specs=[rspec(m0), mspec, mspec, rspec(n_sp),
                  wspec3((n_sp, m0, m1)), wspec3((n_sp, nb0, m1)),
                  wspec2((n_sp, m1)),
                  wspec3((n_sp, m1, nb1)), wspec2((n_sp, nb1))],
        out_specs=[rspec(m1), rspec(nb1)],
        out_shape=[jax.ShapeDtypeStruct((n, m1), jnp.float32),
                   jax.ShapeDtypeStruct((n, nb1), jnp.float32)],
    )
    internal1, neigh1 = k2(internal0, merged0[0], merged0[1], oh,
                           W1a, W1b, b1, Wn1, bn1)

    merged1 = sc_scatter(neigh1, idx_flat, distances, dcoef)

    wfa = Wf[:, :m1, 0].T
    wfb = Wf[:, m1:, 0].T
    bfv = bf.reshape(1, n_sp)
    z2 = lambda i: (0, 0)
    k3 = pl.pallas_call(
        _k3_body,
        grid=(1,),
        in_specs=[pl.BlockSpec((n, m1), z2),
                  pl.BlockSpec((n, nb1), z2),
                  pl.BlockSpec((n, nb1), z2),
                  pl.BlockSpec((n, n_sp), z2),
                  pl.BlockSpec((bsz, na), z2),
                  pl.BlockSpec((m1, n_sp), z2),
                  pl.BlockSpec((nb1, n_sp), z2),
                  pl.BlockSpec((1, n_sp), z2),
                  pl.BlockSpec((bsz, 1), z2)],
        out_specs=[pl.BlockSpec((bsz, na), z2),
                   pl.BlockSpec((bsz, na), z2)],
        out_shape=[jax.ShapeDtypeStruct((bsz, na), jnp.float32),
                   jax.ShapeDtypeStruct((bsz, na), jnp.float32)],
    )
    charges, precharges = k3(internal1, merged1[0], merged1[1], oh, species,
                             wfa, wfb, bfv, total_charges.reshape(bsz, 1))
    return species, charges, precharges


# speedup vs baseline: 8.2957x; 1.1504x over previous
import functools

import jax
import jax.numpy as jnp
from jax import lax
from jax.experimental import pallas as pl
from jax.experimental.pallas import tpu as pltpu
from jax.experimental.pallas import tpu_sc as plsc

RC = 5.2

NC, NS, L = 2, 16, 16
NW = NC * NS

CHUNK = 80

MROWS = 10240
RPT = MROWS // NS


def _celu(x):
    return jnp.where(x > 0, x, jnp.exp(x) - 1.0)


def _expert_linear(x, oh, w_ref, b_ref):
    n_sp = w_ref.shape[0]
    acc = None
    for s in range(n_sp):
        y = jnp.dot(x, w_ref[s], preferred_element_type=jnp.float32)
        y = oh[:, s:s + 1] * (y + b_ref[s][None, :])
        acc = y if acc is None else acc + y
    return acc


def _k1_body(x_ref, oh_ref, w0_ref, b0_ref, wn0_ref, bn0_ref,
             int_ref, ngh_ref):
    x = x_ref[...]
    oh = oh_ref[...]
    internal = _celu(_expert_linear(x, oh, w0_ref, b0_ref))
    int_ref[...] = internal
    ngh_ref[...] = _celu(_expert_linear(internal, oh, wn0_ref, bn0_ref))


def _k2_body(x_ref, ma_ref, mb_ref, oh_ref, w1a_ref, w1b_ref, b1_ref,
             wn1_ref, bn1_ref, int_ref, ngh_ref):
    x = x_ref[...]
    m = ma_ref[...] + mb_ref[...]
    oh = oh_ref[...]
    n_sp = w1a_ref.shape[0]
    acc = None
    for s in range(n_sp):
        y = jnp.dot(x, w1a_ref[s], preferred_element_type=jnp.float32)
        y = y + jnp.dot(m, w1b_ref[s], preferred_element_type=jnp.float32)
        y = oh[:, s:s + 1] * (y + b1_ref[s][None, :])
        acc = y if acc is None else acc + y
    internal = _celu(acc)
    int_ref[...] = internal
    ngh_ref[...] = _celu(_expert_linear(internal, oh, wn1_ref, bn1_ref))


def _k3_body(x_ref, ma_ref, mb_ref, oh_ref, sp_ref, wfa_ref, wfb_ref,
             bf_ref, tq_ref, ch_ref, pc_ref):
    x = x_ref[...]
    m = ma_ref[...] + mb_ref[...]
    p = jnp.dot(x, wfa_ref[...], preferred_element_type=jnp.float32)
    p = p + jnp.dot(m, wfb_ref[...], preferred_element_type=jnp.float32)
    p = p + bf_ref[...]
    prech = jnp.sum(oh_ref[...] * p, axis=1)
    b, a = sp_ref.shape
    prech = prech.reshape(b, a)
    sp = sp_ref[...]
    dummy = sp != -1
    cnt = jnp.sum(dummy.astype(jnp.float32), axis=1, keepdims=True)
    tp = jnp.sum(prech, axis=1, keepdims=True)
    ch = prech + (tq_ref[...] - tp) / cnt
    ch_ref[...] = jnp.where(dummy, ch, 0.0)
    pc_ref[...] = prech


def _sc_scatter_build(p):
    mesh = plsc.VectorSubcoreMesh(core_axis_name="c", subcore_axis_name="s",
                                  num_cores=NC, num_subcores=NS)
    ept = p // NS
    cpt = ept // CHUNK
    assert ept % CHUNK == 0 and CHUNK % 8 == 0
    nbuf = 4
    nidx = 4

    @functools.partial(
        pl.kernel,
        out_type=(jax.ShapeDtypeStruct((MROWS, 128), jnp.float32),
                  jax.ShapeDtypeStruct((MROWS, 128), jnp.float32)),
        mesh=mesh,
        scratch_types=[
            [pltpu.VMEM((CHUNK, 128), jnp.float32)] * nbuf,
            [pltpu.VMEM((CHUNK,), jnp.int32)] * nidx,
            [pltpu.VMEM((CHUNK,), jnp.float32)] * nidx,
            [pltpu.VMEM((CHUNK,), jnp.int32)] * nbuf,
            pltpu.VMEM((2, L), jnp.float32),
            pltpu.VMEM_SHARED((MROWS, 128), jnp.float32),
            [pltpu.SemaphoreType.DMA] * nidx,
            [pltpu.SemaphoreType.DMA] * nbuf,
            [pltpu.SemaphoreType.DMA] * nbuf,
            [pltpu.SemaphoreType.DMA] * nbuf,
        ],
        compiler_params=pltpu.CompilerParams(needs_layout_passes=False),
    )
    def sc_kernel(neigh, idx, dist, dcoef, out0, out1,
                  rows, isrc_c, dist_c, idst_c, dcoef_v, acc,
                  isem, dsem, gsem, ssem):
        c = lax.axis_index("c")
        s = lax.axis_index("s")
        dst_base = c * p + s * ept
        src_base = (1 - c) * p + s * ept
        d_base = s * ept

        def _zrow(r, carry):
            for g in range(8):
                rows[0][r, pl.ds(g * L, L)] = jnp.zeros((L,), jnp.float32)
            return carry
        lax.fori_loop(0, CHUNK, _zrow, 0)
        for k in range(RPT // CHUNK):
            pltpu.sync_copy(rows[0],
                            acc.at[pl.ds(s * RPT + k * CHUNK, CHUNK)])

        pltpu.sync_copy(dcoef, dcoef_v)
        dp2 = dcoef_v[0, :]
        df2 = dcoef_v[1, :]

        plsc.subcore_barrier()

        def _fire_src(j, si):
            pltpu.async_copy(idx.at[pl.ds(src_base + j * CHUNK, CHUNK)],
                             isrc_c[si], isem[si])
            pltpu.async_copy(dist.at[pl.ds(d_base + j * CHUNK, CHUNK)],
                             dist_c[si], isem[si])

        def _wait_src(j, si):
            pltpu.make_async_copy(idx.at[pl.ds(src_base + j * CHUNK, CHUNK)],
                                  isrc_c[si], isem[si]).wait()
            pltpu.make_async_copy(dist.at[pl.ds(d_base + j * CHUNK, CHUNK)],
                                  dist_c[si], isem[si]).wait()

        def _make_step(S, Sg, SiA, SiB, SiP):
            def step(j, drain):
                @pl.when(j + 3 < cpt)
                def _():
                    _fire_src(j + 3, SiA)

                @pl.when(j + 2 < cpt)
                def _():
                    if drain:
                        pltpu.make_async_copy(
                            rows[Sg], acc.at[idst_c[Sg]], ssem[Sg]).wait()
                    _wait_src(j + 2, SiB)
                    pltpu.async_copy(neigh.at[isrc_c[SiB]],
                                     rows[Sg], gsem[Sg])
                    pltpu.async_copy(
                        idx.at[pl.ds(dst_base + (j + 2) * CHUNK, CHUNK)],
                        idst_c[Sg], dsem[Sg])

                pltpu.make_async_copy(neigh.at[isrc_c[SiP]],
                                      rows[S], gsem[S]).wait()
                dbuf = dist_c[SiP]
                for g in range(CHUNK // L):
                    d = dbuf[pl.ds(g * L, L)]
                    x = d * (1.0 / RC)
                    x2 = jnp.clip(x * x, 0.0, 1.0 - 1e-6)
                    f = jnp.exp(1.0 - 1.0 / (1.0 - x2))
                    dec = jnp.where(d < RC, f, 0.0)
                    dbuf[pl.ds(g * L, L)] = dp2 * jnp.exp(-df2 * d) * dec
                def _scale8(t, carry2):
                    for u in range(8):
                        e = t * 8 + u
                        bc = plsc.load_gather(
                            dbuf, [jnp.full((L,), e, jnp.int32)])
                        for q in range(8):
                            rows[S][e, pl.ds(q * L, L)] = (
                                rows[S][e, pl.ds(q * L, L)] * bc)
                    return carry2
                lax.fori_loop(0, CHUNK // 8, _scale8, 0)
                pltpu.make_async_copy(
                    idx.at[pl.ds(dst_base + 0 * CHUNK, CHUNK)],
                    idst_c[S], dsem[S]).wait()
                pltpu.async_copy(rows[S], acc.at[idst_c[S]],
                                 ssem[S], add=True)
            return step

        steps = [_make_step(b % nbuf, (b + 2) % nbuf,
                            (b + 3) % nidx, (b + 2) % nidx, b % nidx)
                 for b in range(nbuf)]

        for q in range(3):
            _fire_src(q, q)
        for q in range(2):
            _wait_src(q, q)
            pltpu.async_copy(neigh.at[isrc_c[q]], rows[q], gsem[q])
            pltpu.async_copy(idx.at[pl.ds(dst_base + q * CHUNK, CHUNK)],
                             idst_c[q], dsem[q])
        for b in range(nbuf):
            steps[b](b, drain=(b >= 2))

        def _cycle(r, carry):
            for b in range(nbuf):
                steps[b](r * nbuf + b, drain=True)
            return carry
        lax.fori_loop(1, cpt // nbuf, _cycle, 0)

        for t in range((cpt // nbuf) * nbuf, cpt):
            steps[t % nbuf](t, drain=True)
        for b in range(nbuf):
            pltpu.make_async_copy(
                rows[b], acc.at[idst_c[b]], ssem[b]).wait()

        plsc.subcore_barrier()

        @pl.when(c == 0)
        def _():
            pltpu.sync_copy(acc.at[pl.ds(s * RPT, RPT)],
                            out0.at[pl.ds(s * RPT, RPT)])

        @pl.when(c == 1)
        def _():
            pltpu.sync_copy(acc.at[pl.ds(s * RPT, RPT)],
                            out1.at[pl.ds(s * RPT, RPT)])

    return sc_kernel


def kernel(species, aev, atom_index12, distances, total_charges,
           W0, b0, Wn0, bn0, W1, b1, Wn1, bn1, Wf, bf,
           decay_prefactor, decay_factor):
    bsz, na = species.shape
    n = bsz * na
    d_aev = aev.shape[-1]
    n_sp = W0.shape[0]
    m0 = W0.shape[-1]
    nb0 = Wn0.shape[-1]
    m1 = W1.shape[-1]
    nb1 = Wn1.shape[-1]
    p = atom_index12.shape[1]

    species_ = species.reshape(-1)
    feats = aev.reshape(n, d_aev)
    oh = (species_[:, None] == jnp.arange(n_sp, dtype=species_.dtype)[None, :]
          ).astype(jnp.float32)

    idx_flat = atom_index12.reshape(-1).astype(jnp.int32)
    dcoef = jnp.stack([
        jnp.full((L,), decay_prefactor.astype(jnp.float32) ** 2),
        jnp.full((L,), decay_factor.astype(jnp.float32) ** 2)])

    sc_scatter = _sc_scatter_build(p)

    rows = 400
    grid = (n // rows,)
    wspec3 = lambda shp: pl.BlockSpec(shp, lambda i: (0, 0, 0))
    wspec2 = lambda shp: pl.BlockSpec(shp, lambda i: (0, 0))
    rspec = lambda width: pl.BlockSpec((rows, width), lambda i: (i, 0))

    k1 = pl.pallas_call(
        _k1_body,
        grid=grid,
        in_specs=[rspec(d_aev), rspec(n_sp),
                  wspec3((n_sp, d_aev, m0)), wspec2((n_sp, m0)),
                  wspec3((n_sp, m0, nb0)), wspec2((n_sp, nb0))],
        out_specs=[rspec(m0), rspec(nb0)],
        out_shape=[jax.ShapeDtypeStruct((n, m0), jnp.float32),
                   jax.ShapeDtypeStruct((n, nb0), jnp.float32)],
    )
    internal0, neigh0 = k1(feats, oh, W0, b0, Wn0, bn0)

    merged0a, merged0b = sc_scatter(neigh0, idx_flat, distances, dcoef)

    W1a = W1[:, :m0, :]
    W1b = W1[:, m0:, :]
    mspec = pl.BlockSpec((rows, nb0), lambda i: (i, 0))
    k2 = pl.pallas_call(
        _k2_body,
        grid=grid,
        in_specs=[rspec(m0), mspec, mspec, rspec(n_sp),
                  wspec3((n_sp, m0, m1)), wspec3((n_sp, nb0, m1)),
                  wspec2((n_sp, m1)),
                  wspec3((n_sp, m1, nb1)), wspec2((n_sp, nb1))],
        out_specs=[rspec(m1), rspec(nb1)],
        out_shape=[jax.ShapeDtypeStruct((n, m1), jnp.float32),
                   jax.ShapeDtypeStruct((n, nb1), jnp.float32)],
    )
    internal1, neigh1 = k2(internal0, merged0a, merged0b, oh,
                           W1a, W1b, b1, Wn1, bn1)

    merged1a, merged1b = sc_scatter(neigh1, idx_flat, distances, dcoef)

    wfa = Wf[:, :m1, 0].T
    wfb = Wf[:, m1:, 0].T
    bfv = bf.reshape(1, n_sp)
    z2 = lambda i: (0, 0)
    k3 = pl.pallas_call(
        _k3_body,
        grid=(1,),
        in_specs=[pl.BlockSpec((n, m1), z2),
                  pl.BlockSpec((n, nb1), z2),
                  pl.BlockSpec((n, nb1), z2),
                  pl.BlockSpec((n, n_sp), z2),
                  pl.BlockSpec((bsz, na), z2),
                  pl.BlockSpec((m1, n_sp), z2),
                  pl.BlockSpec((nb1, n_sp), z2),
                  pl.BlockSpec((1, n_sp), z2),
                  pl.BlockSpec((bsz, 1), z2)],
        out_specs=[pl.BlockSpec((bsz, na), z2),
                   pl.BlockSpec((bsz, na), z2)],
        out_shape=[jax.ShapeDtypeStruct((bsz, na), jnp.float32),
                   jax.ShapeDtypeStruct((bsz, na), jnp.float32)],
    )
    charges, precharges = k3(internal1, merged1a, merged1b, oh, species,
                             wfa, wfb, bfv, total_charges.reshape(bsz, 1))
    return species, charges, precharges


# speedup vs baseline: 8.4354x; 1.0168x over previous
import functools

import jax
import jax.numpy as jnp
from jax import lax
from jax.experimental import pallas as pl
from jax.experimental.pallas import tpu as pltpu
from jax.experimental.pallas import tpu_sc as plsc

RC = 5.2

NC, NS, L = 2, 16, 16
NW = NC * NS

CHUNK = 80

MROWS = 10240
RPT = MROWS // NS


def _celu(x):
    return jnp.where(x > 0, x, jnp.exp(x) - 1.0)


def _expert_linear(x, oh, w_ref, b_ref):
    n_sp = w_ref.shape[0]
    acc = None
    for s in range(n_sp):
        y = jnp.dot(x, w_ref[s], preferred_element_type=jnp.float32)
        y = oh[:, s:s + 1] * (y + b_ref[s][None, :])
        acc = y if acc is None else acc + y
    return acc


def _k1_body(x_ref, oh_ref, w0_ref, b0_ref, wn0_ref, bn0_ref,
             int_ref, ngh_ref):
    x = x_ref[...]
    oh = oh_ref[...]
    internal = _celu(_expert_linear(x, oh, w0_ref, b0_ref))
    int_ref[...] = internal
    ngh_ref[...] = _celu(_expert_linear(internal, oh, wn0_ref, bn0_ref))


def _k2_body(x_ref, ma_ref, mb_ref, oh_ref, w1a_ref, w1b_ref, b1_ref,
             wn1_ref, bn1_ref, int_ref, ngh_ref):
    x = x_ref[...]
    m = ma_ref[...] + mb_ref[...]
    oh = oh_ref[...]
    n_sp = w1a_ref.shape[0]
    acc = None
    for s in range(n_sp):
        y = jnp.dot(x, w1a_ref[s], preferred_element_type=jnp.float32)
        y = y + jnp.dot(m, w1b_ref[s], preferred_element_type=jnp.float32)
        y = oh[:, s:s + 1] * (y + b1_ref[s][None, :])
        acc = y if acc is None else acc + y
    internal = _celu(acc)
    int_ref[...] = internal
    ngh_ref[...] = _celu(_expert_linear(internal, oh, wn1_ref, bn1_ref))


def _k3_body(x_ref, ma_ref, mb_ref, oh_ref, sp_ref, wfa_ref, wfb_ref,
             bf_ref, tq_ref, ch_ref, pc_ref):
    x = x_ref[...]
    m = ma_ref[...] + mb_ref[...]
    p = jnp.dot(x, wfa_ref[...], preferred_element_type=jnp.float32)
    p = p + jnp.dot(m, wfb_ref[...], preferred_element_type=jnp.float32)
    p = p + bf_ref[...]
    prech = jnp.sum(oh_ref[...] * p, axis=1)
    b, a = sp_ref.shape
    prech = prech.reshape(b, a)
    sp = sp_ref[...]
    dummy = sp != -1
    cnt = jnp.sum(dummy.astype(jnp.float32), axis=1, keepdims=True)
    tp = jnp.sum(prech, axis=1, keepdims=True)
    ch = prech + (tq_ref[...] - tp) / cnt
    ch_ref[...] = jnp.where(dummy, ch, 0.0)
    pc_ref[...] = prech


def _sc_scatter_build(p):
    mesh = plsc.VectorSubcoreMesh(core_axis_name="c", subcore_axis_name="s",
                                  num_cores=NC, num_subcores=NS)
    ept = p // NS
    cpt = ept // CHUNK
    assert ept % CHUNK == 0 and CHUNK % 8 == 0
    nbuf = 4
    nidx = 4

    @functools.partial(
        pl.kernel,
        out_type=(jax.ShapeDtypeStruct((MROWS, 128), jnp.float32),
                  jax.ShapeDtypeStruct((MROWS, 128), jnp.float32)),
        mesh=mesh,
        scratch_types=[
            [pltpu.VMEM((CHUNK, 128), jnp.float32)] * nbuf,
            [pltpu.VMEM((CHUNK,), jnp.int32)] * nidx,
            [pltpu.VMEM((CHUNK,), jnp.float32)] * nidx,
            [pltpu.VMEM((CHUNK,), jnp.int32)] * nbuf,
            pltpu.VMEM((2, L), jnp.float32),
            pltpu.VMEM_SHARED((MROWS, 128), jnp.float32),
            [pltpu.SemaphoreType.DMA] * nidx,
            [pltpu.SemaphoreType.DMA] * nbuf,
            [pltpu.SemaphoreType.DMA] * nbuf,
            [pltpu.SemaphoreType.DMA] * nbuf,
        ],
        compiler_params=pltpu.CompilerParams(needs_layout_passes=False),
    )
    def sc_kernel(neigh, idx, dist, dcoef, out0, out1,
                  rows, isrc_c, dist_c, idst_c, dcoef_v, acc,
                  isem, dsem, gsem, ssem):
        c = lax.axis_index("c")
        s = lax.axis_index("s")
        dst_base = c * p + s * ept
        src_base = (1 - c) * p + s * ept
        d_base = s * ept

        def _zrow(r, carry):
            for g in range(8):
                rows[0][r, pl.ds(g * L, L)] = jnp.zeros((L,), jnp.float32)
            return carry
        lax.fori_loop(0, CHUNK, _zrow, 0)
        for k in range(RPT // CHUNK):
            pltpu.sync_copy(rows[0],
                            acc.at[pl.ds(s * RPT + k * CHUNK, CHUNK)])

        pltpu.sync_copy(dcoef, dcoef_v)
        dp2 = dcoef_v[0, :]
        df2 = dcoef_v[1, :]

        plsc.subcore_barrier()

        def _fire_src(j, si):
            pltpu.async_copy(idx.at[pl.ds(src_base + j * CHUNK, CHUNK)],
                             isrc_c[si], isem[si])
            pltpu.async_copy(dist.at[pl.ds(d_base + j * CHUNK, CHUNK)],
                             dist_c[si], isem[si])

        def _wait_src(j, si):
            pltpu.make_async_copy(idx.at[pl.ds(src_base + j * CHUNK, CHUNK)],
                                  isrc_c[si], isem[si]).wait()
            pltpu.make_async_copy(dist.at[pl.ds(d_base + j * CHUNK, CHUNK)],
                                  dist_c[si], isem[si]).wait()

        def _make_step(S, Sg, SiA, SiB, SiP):
            def step(j, drain):
                @pl.when(j + 3 < cpt)
                def _():
                    _fire_src(j + 3, SiA)

                @pl.when(j + 2 < cpt)
                def _():
                    if drain:
                        pltpu.make_async_copy(
                            rows[Sg], acc.at[idst_c[Sg]], ssem[Sg]).wait()
                    _wait_src(j + 2, SiB)
                    pltpu.async_copy(neigh.at[isrc_c[SiB]],
                                     rows[Sg], gsem[Sg])
                    pltpu.async_copy(
                        idx.at[pl.ds(dst_base + (j + 2) * CHUNK, CHUNK)],
                        idst_c[Sg], dsem[Sg])

                pltpu.make_async_copy(neigh.at[isrc_c[SiP]],
                                      rows[S], gsem[S]).wait()
                dbuf = dist_c[SiP]
                for g in range(CHUNK // L):
                    d = dbuf[pl.ds(g * L, L)]
                    x = d * (1.0 / RC)
                    x2 = jnp.clip(x * x, 0.0, 1.0 - 1e-6)
                    f = jnp.exp(1.0 - 1.0 / (1.0 - x2))
                    dec = jnp.where(d < RC, f, 0.0)
                    dbuf[pl.ds(g * L, L)] = dp2 * jnp.exp(-df2 * d) * dec
                def _scale8(t, carry2):
                    for u in range(0, 8, 2):
                        e0 = t * 8 + u
                        e1 = t * 8 + u + 1
                        bc0 = plsc.load_gather(
                            dbuf, [jnp.full((L,), e0, jnp.int32)])
                        bc1 = plsc.load_gather(
                            dbuf, [jnp.full((L,), e1, jnp.int32)])
                        for q in range(8):
                            rows[S][e0, pl.ds(q * L, L)] = (
                                rows[S][e0, pl.ds(q * L, L)] * bc0)
                            rows[S][e1, pl.ds(q * L, L)] = (
                                rows[S][e1, pl.ds(q * L, L)] * bc1)
                    return carry2
                lax.fori_loop(0, CHUNK // 8, _scale8, 0)
                pltpu.make_async_copy(
                    idx.at[pl.ds(dst_base + 0 * CHUNK, CHUNK)],
                    idst_c[S], dsem[S]).wait()
                pltpu.async_copy(rows[S], acc.at[idst_c[S]],
                                 ssem[S], add=True)
            return step

        steps = [_make_step(b % nbuf, (b + 2) % nbuf,
                            (b + 3) % nidx, (b + 2) % nidx, b % nidx)
                 for b in range(nbuf)]

        for q in range(3):
            _fire_src(q, q)
        for q in range(2):
            _wait_src(q, q)
            pltpu.async_copy(neigh.at[isrc_c[q]], rows[q], gsem[q])
            pltpu.async_copy(idx.at[pl.ds(dst_base + q * CHUNK, CHUNK)],
                             idst_c[q], dsem[q])
        for b in range(nbuf):
            steps[b](b, drain=(b >= 2))

        def _cycle(r, carry):
            for b in range(nbuf):
                steps[b](r * nbuf + b, drain=True)
            return carry
        lax.fori_loop(1, cpt // nbuf, _cycle, 0)

        for t in range((cpt // nbuf) * nbuf, cpt):
            steps[t % nbuf](t, drain=True)
        for b in range(nbuf):
            pltpu.make_async_copy(
                rows[b], acc.at[idst_c[b]], ssem[b]).wait()

        plsc.subcore_barrier()

        @pl.when(c == 0)
        def _():
            pltpu.sync_copy(acc.at[pl.ds(s * RPT, RPT)],
                            out0.at[pl.ds(s * RPT, RPT)])

        @pl.when(c == 1)
        def _():
            pltpu.sync_copy(acc.at[pl.ds(s * RPT, RPT)],
                            out1.at[pl.ds(s * RPT, RPT)])

    return sc_kernel


def kernel(species, aev, atom_index12, distances, total_charges,
           W0, b0, Wn0, bn0, W1, b1, Wn1, bn1, Wf, bf,
           decay_prefactor, decay_factor):
    bsz, na = species.shape
    n = bsz * na
    d_aev = aev.shape[-1]
    n_sp = W0.shape[0]
    m0 = W0.shape[-1]
    nb0 = Wn0.shape[-1]
    m1 = W1.shape[-1]
    nb1 = Wn1.shape[-1]
    p = atom_index12.shape[1]

    species_ = species.reshape(-1)
    feats = aev.reshape(n, d_aev)
    oh = (species_[:, None] == jnp.arange(n_sp, dtype=species_.dtype)[None, :]
          ).astype(jnp.float32)

    idx_flat = atom_index12.reshape(-1).astype(jnp.int32)
    dcoef = jnp.stack([
        jnp.full((L,), decay_prefactor.astype(jnp.float32) ** 2),
        jnp.full((L,), decay_factor.astype(jnp.float32) ** 2)])

    sc_scatter = _sc_scatter_build(p)

    rows = 400
    grid = (n // rows,)
    wspec3 = lambda shp: pl.BlockSpec(shp, lambda i: (0, 0, 0))
    wspec2 = lambda shp: pl.BlockSpec(shp, lambda i: (0, 0))
    rspec = lambda width: pl.BlockSpec((rows, width), lambda i: (i, 0))

    k1 = pl.pallas_call(
        _k1_body,
        grid=grid,
        in_specs=[rspec(d_aev), rspec(n_sp),
                  wspec3((n_sp, d_aev, m0)), wspec2((n_sp, m0)),
                  wspec3((n_sp, m0, nb0)), wspec2((n_sp, nb0))],
        out_specs=[rspec(m0), rspec(nb0)],
        out_shape=[jax.ShapeDtypeStruct((n, m0), jnp.float32),
                   jax.ShapeDtypeStruct((n, nb0), jnp.float32)],
    )
    internal0, neigh0 = k1(feats, oh, W0, b0, Wn0, bn0)

    merged0a, merged0b = sc_scatter(neigh0, idx_flat, distances, dcoef)

    W1a = W1[:, :m0, :]
    W1b = W1[:, m0:, :]
    mspec = pl.BlockSpec((rows, nb0), lambda i: (i, 0))
    k2 = pl.pallas_call(
        _k2_body,
        grid=grid,
        in_specs=[rspec(m0), mspec, mspec, rspec(n_sp),
                  wspec3((n_sp, m0, m1)), wspec3((n_sp, nb0, m1)),
                  wspec2((n_sp, m1)),
                  wspec3((n_sp, m1, nb1)), wspec2((n_sp, nb1))],
        out_specs=[rspec(m1), rspec(nb1)],
        out_shape=[jax.ShapeDtypeStruct((n, m1), jnp.float32),
                   jax.ShapeDtypeStruct((n, nb1), jnp.float32)],
    )
    internal1, neigh1 = k2(internal0, merged0a, merged0b, oh,
                           W1a, W1b, b1, Wn1, bn1)

    merged1a, merged1b = sc_scatter(neigh1, idx_flat, distances, dcoef)

    wfa = Wf[:, :m1, 0].T
    wfb = Wf[:, m1:, 0].T
    bfv = bf.reshape(1, n_sp)
    z2 = lambda i: (0, 0)
    k3 = pl.pallas_call(
        _k3_body,
        grid=(1,),
        in_specs=[pl.BlockSpec((n, m1), z2),
                  pl.BlockSpec((n, nb1), z2),
                  pl.BlockSpec((n, nb1), z2),
                  pl.BlockSpec((n, n_sp), z2),
                  pl.BlockSpec((bsz, na), z2),
                  pl.BlockSpec((m1, n_sp), z2),
                  pl.BlockSpec((nb1, n_sp), z2),
                  pl.BlockSpec((1, n_sp), z2),
                  pl.BlockSpec((bsz, 1), z2)],
        out_specs=[pl.BlockSpec((bsz, na), z2),
                   pl.BlockSpec((bsz, na), z2)],
        out_shape=[jax.ShapeDtypeStruct((bsz, na), jnp.float32),
                   jax.ShapeDtypeStruct((bsz, na), jnp.float32)],
    )
    charges, precharges = k3(internal1, merged1a, merged1b, oh, species,
                             wfa, wfb, bfv, total_charges.reshape(bsz, 1))
    return species, charges, precharges


# speedup vs baseline: 8.4770x; 1.0049x over previous
import functools

import jax
import jax.numpy as jnp
from jax import lax
from jax.experimental import pallas as pl
from jax.experimental.pallas import tpu as pltpu
from jax.experimental.pallas import tpu_sc as plsc

RC = 5.2

NC, NS, L = 2, 16, 16
NW = NC * NS

CHUNK = 80

MROWS = 10240
RPT = MROWS // NS


def _celu(x):
    return jnp.where(x > 0, x, jnp.exp(x) - 1.0)


def _expert_linear(x, oh, w_ref, b_ref):
    n_sp = w_ref.shape[0]
    acc = None
    for s in range(n_sp):
        y = jnp.dot(x, w_ref[s], preferred_element_type=jnp.float32)
        y = oh[:, s:s + 1] * (y + b_ref[s][None, :])
        acc = y if acc is None else acc + y
    return acc


def _k1_body(x_ref, oh_ref, w0_ref, b0_ref, wn0_ref, bn0_ref,
             int_ref, ngh_ref):
    x = x_ref[...]
    oh = oh_ref[...]
    internal = _celu(_expert_linear(x, oh, w0_ref, b0_ref))
    int_ref[...] = internal
    ngh_ref[...] = _celu(_expert_linear(internal, oh, wn0_ref, bn0_ref))


def _k2a_body(x_ref, oh_ref, w1a_ref, b1_ref, ha_ref):
    ha_ref[...] = _expert_linear(x_ref[...], oh_ref[...], w1a_ref, b1_ref)


def _k2b_body(ha_ref, ma_ref, mb_ref, oh_ref, w1b_ref,
              wn1_ref, bn1_ref, int_ref, ngh_ref):
    m = ma_ref[...] + mb_ref[...]
    oh = oh_ref[...]
    n_sp = w1b_ref.shape[0]
    acc = ha_ref[...]
    for s in range(n_sp):
        y = jnp.dot(m, w1b_ref[s], preferred_element_type=jnp.float32)
        acc = acc + oh[:, s:s + 1] * y
    internal = _celu(acc)
    int_ref[...] = internal
    ngh_ref[...] = _celu(_expert_linear(internal, oh, wn1_ref, bn1_ref))


def _k3_body(x_ref, ma_ref, mb_ref, oh_ref, sp_ref, wfa_ref, wfb_ref,
             bf_ref, tq_ref, ch_ref, pc_ref):
    x = x_ref[...]
    m = ma_ref[...] + mb_ref[...]
    p = jnp.dot(x, wfa_ref[...], preferred_element_type=jnp.float32)
    p = p + jnp.dot(m, wfb_ref[...], preferred_element_type=jnp.float32)
    p = p + bf_ref[...]
    prech = jnp.sum(oh_ref[...] * p, axis=1)
    b, a = sp_ref.shape
    prech = prech.reshape(b, a)
    sp = sp_ref[...]
    dummy = sp != -1
    cnt = jnp.sum(dummy.astype(jnp.float32), axis=1, keepdims=True)
    tp = jnp.sum(prech, axis=1, keepdims=True)
    ch = prech + (tq_ref[...] - tp) / cnt
    ch_ref[...] = jnp.where(dummy, ch, 0.0)
    pc_ref[...] = prech


def _sc_scatter_build(p):
    mesh = plsc.VectorSubcoreMesh(core_axis_name="c", subcore_axis_name="s",
                                  num_cores=NC, num_subcores=NS)
    ept = p // NS
    cpt = ept // CHUNK
    assert ept % CHUNK == 0 and CHUNK % 8 == 0
    nbuf = 4
    nidx = 4

    @functools.partial(
        pl.kernel,
        out_type=(jax.ShapeDtypeStruct((MROWS, 128), jnp.float32),
                  jax.ShapeDtypeStruct((MROWS, 128), jnp.float32)),
        mesh=mesh,
        scratch_types=[
            [pltpu.VMEM((CHUNK, 128), jnp.float32)] * nbuf,
            [pltpu.VMEM((CHUNK,), jnp.int32)] * nidx,
            [pltpu.VMEM((CHUNK,), jnp.float32)] * nidx,
            [pltpu.VMEM((CHUNK,), jnp.int32)] * nbuf,
            pltpu.VMEM((2, L), jnp.float32),
            pltpu.VMEM_SHARED((MROWS, 128), jnp.float32),
            [pltpu.SemaphoreType.DMA] * nidx,
            [pltpu.SemaphoreType.DMA] * nbuf,
            [pltpu.SemaphoreType.DMA] * nbuf,
            [pltpu.SemaphoreType.DMA] * nbuf,
        ],
        compiler_params=pltpu.CompilerParams(needs_layout_passes=False),
    )
    def sc_kernel(neigh, idx, dist, dcoef, out0, out1,
                  rows, isrc_c, dist_c, idst_c, dcoef_v, acc,
                  isem, dsem, gsem, ssem):
        c = lax.axis_index("c")
        s = lax.axis_index("s")
        dst_base = c * p + s * ept
        src_base = (1 - c) * p + s * ept
        d_base = s * ept

        def _zrow(r, carry):
            for g in range(8):
                rows[0][r, pl.ds(g * L, L)] = jnp.zeros((L,), jnp.float32)
            return carry
        lax.fori_loop(0, CHUNK, _zrow, 0)
        for k in range(RPT // CHUNK):
            pltpu.sync_copy(rows[0],
                            acc.at[pl.ds(s * RPT + k * CHUNK, CHUNK)])

        pltpu.sync_copy(dcoef, dcoef_v)
        dp2 = dcoef_v[0, :]
        df2 = dcoef_v[1, :]

        plsc.subcore_barrier()

        def _fire_src(j, si):
            pltpu.async_copy(idx.at[pl.ds(src_base + j * CHUNK, CHUNK)],
                             isrc_c[si], isem[si])
            pltpu.async_copy(dist.at[pl.ds(d_base + j * CHUNK, CHUNK)],
                             dist_c[si], isem[si])

        def _wait_src(j, si):
            pltpu.make_async_copy(idx.at[pl.ds(src_base + j * CHUNK, CHUNK)],
                                  isrc_c[si], isem[si]).wait()
            pltpu.make_async_copy(dist.at[pl.ds(d_base + j * CHUNK, CHUNK)],
                                  dist_c[si], isem[si]).wait()

        def _make_step(S, Sg, SiA, SiB, SiP):
            def step(j, drain):
                @pl.when(j + 3 < cpt)
                def _():
                    _fire_src(j + 3, SiA)

                @pl.when(j + 2 < cpt)
                def _():
                    if drain:
                        pltpu.make_async_copy(
                            rows[Sg], acc.at[idst_c[Sg]], ssem[Sg]).wait()
                    _wait_src(j + 2, SiB)
                    pltpu.async_copy(neigh.at[isrc_c[SiB]],
                                     rows[Sg], gsem[Sg])
                    pltpu.async_copy(
                        idx.at[pl.ds(dst_base + (j + 2) * CHUNK, CHUNK)],
                        idst_c[Sg], dsem[Sg])

                pltpu.make_async_copy(neigh.at[isrc_c[SiP]],
                                      rows[S], gsem[S]).wait()
                dbuf = dist_c[SiP]
                for g in range(CHUNK // L):
                    d = dbuf[pl.ds(g * L, L)]
                    x = d * (1.0 / RC)
                    x2 = jnp.clip(x * x, 0.0, 1.0 - 1e-6)
                    f = jnp.exp(1.0 - 1.0 / (1.0 - x2))
                    dec = jnp.where(d < RC, f, 0.0)
                    dbuf[pl.ds(g * L, L)] = dp2 * jnp.exp(-df2 * d) * dec
                def _scale8(t, carry2):
                    for u in range(0, 8, 2):
                        e0 = t * 8 + u
                        e1 = t * 8 + u + 1
                        bc0 = plsc.load_gather(
                            dbuf, [jnp.full((L,), e0, jnp.int32)])
                        bc1 = plsc.load_gather(
                            dbuf, [jnp.full((L,), e1, jnp.int32)])
                        for q in range(8):
                            rows[S][e0, pl.ds(q * L, L)] = (
                                rows[S][e0, pl.ds(q * L, L)] * bc0)
                            rows[S][e1, pl.ds(q * L, L)] = (
                                rows[S][e1, pl.ds(q * L, L)] * bc1)
                    return carry2
                lax.fori_loop(0, CHUNK // 8, _scale8, 0)
                pltpu.make_async_copy(
                    idx.at[pl.ds(dst_base + 0 * CHUNK, CHUNK)],
                    idst_c[S], dsem[S]).wait()
                pltpu.async_copy(rows[S], acc.at[idst_c[S]],
                                 ssem[S], add=True)
            return step

        steps = [_make_step(b % nbuf, (b + 2) % nbuf,
                            (b + 3) % nidx, (b + 2) % nidx, b % nidx)
                 for b in range(nbuf)]

        for q in range(3):
            _fire_src(q, q)
        for q in range(2):
            _wait_src(q, q)
            pltpu.async_copy(neigh.at[isrc_c[q]], rows[q], gsem[q])
            pltpu.async_copy(idx.at[pl.ds(dst_base + q * CHUNK, CHUNK)],
                             idst_c[q], dsem[q])
        for b in range(nbuf):
            steps[b](b, drain=(b >= 2))

        def _cycle(r, carry):
            for b in range(nbuf):
                steps[b](r * nbuf + b, drain=True)
            return carry
        lax.fori_loop(1, cpt // nbuf, _cycle, 0)

        for t in range((cpt // nbuf) * nbuf, cpt):
            steps[t % nbuf](t, drain=True)
        for b in range(nbuf):
            pltpu.make_async_copy(
                rows[b], acc.at[idst_c[b]], ssem[b]).wait()

        plsc.subcore_barrier()

        @pl.when(c == 0)
        def _():
            pltpu.sync_copy(acc.at[pl.ds(s * RPT, RPT)],
                            out0.at[pl.ds(s * RPT, RPT)])

        @pl.when(c == 1)
        def _():
            pltpu.sync_copy(acc.at[pl.ds(s * RPT, RPT)],
                            out1.at[pl.ds(s * RPT, RPT)])

    return sc_kernel


def kernel(species, aev, atom_index12, distances, total_charges,
           W0, b0, Wn0, bn0, W1, b1, Wn1, bn1, Wf, bf,
           decay_prefactor, decay_factor):
    bsz, na = species.shape
    n = bsz * na
    d_aev = aev.shape[-1]
    n_sp = W0.shape[0]
    m0 = W0.shape[-1]
    nb0 = Wn0.shape[-1]
    m1 = W1.shape[-1]
    nb1 = Wn1.shape[-1]
    p = atom_index12.shape[1]

    species_ = species.reshape(-1)
    feats = aev.reshape(n, d_aev)
    oh = (species_[:, None] == jnp.arange(n_sp, dtype=species_.dtype)[None, :]
          ).astype(jnp.float32)

    idx_flat = atom_index12.reshape(-1).astype(jnp.int32)
    dcoef = jnp.stack([
        jnp.full((L,), decay_prefactor.astype(jnp.float32) ** 2),
        jnp.full((L,), decay_factor.astype(jnp.float32) ** 2)])

    sc_scatter = _sc_scatter_build(p)

    rows = 400
    grid = (n // rows,)
    wspec3 = lambda shp: pl.BlockSpec(shp, lambda i: (0, 0, 0))
    wspec2 = lambda shp: pl.BlockSpec(shp, lambda i: (0, 0))
    rspec = lambda width: pl.BlockSpec((rows, width), lambda i: (i, 0))

    k1 = pl.pallas_call(
        _k1_body,
        grid=grid,
        in_specs=[rspec(d_aev), rspec(n_sp),
                  wspec3((n_sp, d_aev, m0)), wspec2((n_sp, m0)),
                  wspec3((n_sp, m0, nb0)), wspec2((n_sp, nb0))],
        out_specs=[rspec(m0), rspec(nb0)],
        out_shape=[jax.ShapeDtypeStruct((n, m0), jnp.float32),
                   jax.ShapeDtypeStruct((n, nb0), jnp.float32)],
    )
    internal0, neigh0 = k1(feats, oh, W0, b0, Wn0, bn0)

    merged0a, merged0b = sc_scatter(neigh0, idx_flat, distances, dcoef)

    W1a = W1[:, :m0, :]
    W1b = W1[:, m0:, :]
    mspec = pl.BlockSpec((rows, nb0), lambda i: (i, 0))
    k2a = pl.pallas_call(
        _k2a_body,
        grid=grid,
        in_specs=[rspec(m0), rspec(n_sp),
                  wspec3((n_sp, m0, m1)), wspec2((n_sp, m1))],
        out_specs=rspec(m1),
        out_shape=jax.ShapeDtypeStruct((n, m1), jnp.float32),
    )
    h_a = k2a(internal0, oh, W1a, b1)
    k2b = pl.pallas_call(
        _k2b_body,
        grid=grid,
        in_specs=[rspec(m1), mspec, mspec, rspec(n_sp),
                  wspec3((n_sp, nb0, m1)),
                  wspec3((n_sp, m1, nb1)), wspec2((n_sp, nb1))],
        out_specs=[rspec(m1), rspec(nb1)],
        out_shape=[jax.ShapeDtypeStruct((n, m1), jnp.float32),
                   jax.ShapeDtypeStruct((n, nb1), jnp.float32)],
    )
    internal1, neigh1 = k2b(h_a, merged0a, merged0b, oh, W1b, Wn1, bn1)

    merged1a, merged1b = sc_scatter(neigh1, idx_flat, distances, dcoef)

    wfa = Wf[:, :m1, 0].T
    wfb = Wf[:, m1:, 0].T
    bfv = bf.reshape(1, n_sp)
    z2 = lambda i: (0, 0)
    k3 = pl.pallas_call(
        _k3_body,
        grid=(1,),
        in_specs=[pl.BlockSpec((n, m1), z2),
                  pl.BlockSpec((n, nb1), z2),
                  pl.BlockSpec((n, nb1), z2),
                  pl.BlockSpec((n, n_sp), z2),
                  pl.BlockSpec((bsz, na), z2),
                  pl.BlockSpec((m1, n_sp), z2),
                  pl.BlockSpec((nb1, n_sp), z2),
                  pl.BlockSpec((1, n_sp), z2),
                  pl.BlockSpec((bsz, 1), z2)],
        out_specs=[pl.BlockSpec((bsz, na), z2),
                   pl.BlockSpec((bsz, na), z2)],
        out_shape=[jax.ShapeDtypeStruct((bsz, na), jnp.float32),
                   jax.ShapeDtypeStruct((bsz, na), jnp.float32)],
    )
    charges, precharges = k3(internal1, merged1a, merged1b, oh, species,
                             wfa, wfb, bfv, total_charges.reshape(bsz, 1))
    return species, charges, precharges
